# baseline (speedup 1.0000x reference)
import functools

import jax
import jax.numpy as jnp
from jax import lax
from jax.experimental import pallas as pl
from jax.experimental.pallas import tpu as pltpu
from jax.experimental.pallas import tpu_sc as plsc

NC = 2
NS = 16
L = 16
CH = 128
SLAB = 4


def _spmm_body(npad, nrnd, s1, s2, s3, e1, e2, e3, part,
               acc, gi, si, rows, gsem, isem):
    c = lax.axis_index("c")
    s = lax.axis_index("s")
    w = c * NS + s
    rpt = npad // NS

    @pl.loop(0, CH)
    def _zero_rows(i):
        for j in range(rows.shape[2] // L):
            rows[0, i, j * L:(j + 1) * L] = jnp.zeros((L,), jnp.float32)

    for k in range(rpt // CH):
        pltpu.sync_copy(rows.at[0], acc.at[pl.ds(s * rpt + k * CH, CH)])
    plsc.subcore_barrier()

    streams = ((0, e1, s1), (1, e1, s1), (0, e2, s2), (1, e2, s2),
               (0, e3, s3), (1, e3, s3))

    nslabt = nrnd // SLAB
    assert nrnd % SLAB == 0 and nslabt % 2 == 0 and SLAB % 2 == 0
    last = nslabt - 1

    for d, e_ref, s_ref in streams:
        base = w * nslabt

        def _idx_load(m, p, _d=d, _e=e_ref):
            pltpu.async_copy(_e.at[_d, base + m], gi.at[p], isem.at[0])
            pltpu.async_copy(_e.at[1 - _d, base + m], si.at[p], isem.at[1])

        def _idx_wait(_e=e_ref):
            pltpu.make_async_copy(_e.at[0, 0], gi.at[0], isem.at[0]).wait()
            pltpu.make_async_copy(_e.at[0, 0], si.at[0], isem.at[1]).wait()

        _idx_load(0, 0)
        _idx_wait()
        pltpu.async_copy(s_ref.at[gi.at[0, 0]], rows.at[0], gsem.at[0])
        _idx_load(1, 1)

        @pl.loop(0, nslabt // 2)
        def _two_slabs(t, _s=s_ref, _idx_load=_idx_load, _idx_wait=_idx_wait):
            for p in (0, 1):
                m = 2 * t + p
                np_ = 1 - p

                @pl.when(m < last)
                def _():
                    _idx_wait()

                for b in range(SLAB):
                    rb = b % 2
                    if b + 1 < SLAB:
                        pltpu.async_copy(_s.at[gi.at[p, b + 1]],
                                         rows.at[1 - rb], gsem.at[1 - rb])
                    else:
                        @pl.when(m < last)
                        def _():
                            pltpu.async_copy(_s.at[gi.at[np_, 0]],
                                             rows.at[1 - rb],
                                             gsem.at[1 - rb])

                    pltpu.make_async_copy(_s.at[gi.at[p, b]], rows.at[rb],
                                          gsem.at[rb]).wait()
                    pltpu.sync_copy(rows.at[rb], acc.at[si.at[p, b]],
                                    add=True)

                @pl.when(m < last - 1)
                def _():
                    _idx_load(m + 2, p)

    plsc.subcore_barrier()
    pltpu.sync_copy(acc.at[pl.ds(s * rpt, rpt)],
                    part.at[c, pl.ds(s * rpt, rpt)])


def _sc_spmm(s1, s2, s3, e1, e2, e3):
    npad, d_model = s1.shape
    nchunk = e1.shape[1] * SLAB
    nrnd = nchunk // (NC * NS)
    mesh = plsc.VectorSubcoreMesh(core_axis_name="c", subcore_axis_name="s")
    body = functools.partial(_spmm_body, npad, nrnd)
    return pl.kernel(
        body,
        out_type=jax.ShapeDtypeStruct((NC, npad, d_model), jnp.float32),
        mesh=mesh,
        scratch_types=[
            pltpu.VMEM_SHARED((npad, d_model), jnp.float32),
            pltpu.VMEM((2, SLAB, CH), jnp.int32),
            pltpu.VMEM((2, SLAB, CH), jnp.int32),
            pltpu.VMEM((2, CH, d_model), jnp.float32),
            pltpu.SemaphoreType.DMA((2,)),
            pltpu.SemaphoreType.DMA((2,)),
        ],
    )(s1, s2, s3, e1, e2, e3)


def _tc_scaled_support_body(x_ref, w_ref, wb_ref, s1_ref, s2_ref, s3_ref):
    sup = jnp.dot(x_ref[...], w_ref[...], preferred_element_type=jnp.float32)
    s1_ref[...] = wb_ref[0, 0] * sup
    s2_ref[...] = wb_ref[1, 0] * sup
    s3_ref[...] = wb_ref[2, 0] * sup


def _tc_scaled_support(x, w, wb, bm):
    n, d_model = x.shape
    grid = (n // bm,)
    blk = pl.BlockSpec((bm, d_model), lambda i: (i, 0))
    return pl.pallas_call(
        _tc_scaled_support_body,
        grid=grid,
        in_specs=[blk,
                  pl.BlockSpec((d_model, d_model), lambda i: (0, 0)),
                  pl.BlockSpec(memory_space=pltpu.SMEM)],
        out_specs=[blk, blk, blk],
        out_shape=[jax.ShapeDtypeStruct((n, d_model), jnp.float32)] * 3,
    )(x, w, wb)


def _tc_merge_support_body(p_ref, b_ref, w_ref, wb_ref,
                           u_ref, s1_ref, s2_ref, s3_ref):
    u = p_ref[0] + p_ref[1] + b_ref[...]
    u_ref[...] = u
    sup = jnp.dot(u, w_ref[...], preferred_element_type=jnp.float32)
    s1_ref[...] = wb_ref[0, 0] * sup
    s2_ref[...] = wb_ref[1, 0] * sup
    s3_ref[...] = wb_ref[2, 0] * sup


def _tc_merge_support(p, b, w, wb, bm):
    _, n, d_model = p.shape
    grid = (n // bm,)
    blk = pl.BlockSpec((bm, d_model), lambda i: (i, 0))
    return pl.pallas_call(
        _tc_merge_support_body,
        grid=grid,
        in_specs=[pl.BlockSpec((2, bm, d_model), lambda i: (0, i, 0)),
                  pl.BlockSpec((1, d_model), lambda i: (0, 0)),
                  pl.BlockSpec((d_model, d_model), lambda i: (0, 0)),
                  pl.BlockSpec(memory_space=pltpu.SMEM)],
        out_specs=[blk, blk, blk, blk],
        out_shape=[jax.ShapeDtypeStruct((n, d_model), jnp.float32)] * 4,
    )(p, b.reshape(1, d_model), w, wb)


def _tc_final_body(u1_ref, q_ref, b_ref, out_ref):
    out_ref[...] = 0.5 * (u1_ref[...] + q_ref[0] + q_ref[1] + b_ref[...])


def _tc_final(u1, q, b, bm, n):
    _, d_model = u1.shape
    grid = (n // bm,)
    blk = pl.BlockSpec((bm, d_model), lambda i: (i, 0))
    return pl.pallas_call(
        _tc_final_body,
        grid=grid,
        in_specs=[blk,
                  pl.BlockSpec((2, bm, d_model), lambda i: (0, i, 0)),
                  pl.BlockSpec((1, d_model), lambda i: (0, 0))],
        out_specs=blk,
        out_shape=jax.ShapeDtypeStruct((n, d_model), jnp.float32),
    )(u1, q, b.reshape(1, d_model))


def _pad_edges(e, n, nchunk_pad):
    e = e.astype(jnp.int32)
    pad = nchunk_pad * CH - e.shape[1]
    ep = jnp.pad(e, ((0, 0), (0, pad)), constant_values=n)
    return ep.reshape(2, nchunk_pad // SLAB, SLAB, CH)


def kernel(x, edge_index1, edge_index2, edge_index3, weight_b, W1, b1, W2, b2):
    n, d_model = x.shape
    npad = ((n + NS * CH - 1) // (NS * CH)) * (NS * CH)
    e = edge_index1.shape[1]
    echunk = NC * NS * 2 * SLAB * CH
    nchunk_pad = ((e + echunk - 1) // echunk) * echunk // CH
    e1 = _pad_edges(edge_index1, n, nchunk_pad)
    e2 = _pad_edges(edge_index2, n, nchunk_pad)
    e3 = _pad_edges(edge_index3, n, nchunk_pad)
    x_pad = jnp.pad(x, ((0, npad - n), (0, 0)))

    s1, s2, s3 = _tc_scaled_support(x_pad, W1, weight_b, npad // 16)
    p = _sc_spmm(s1, s2, s3, e1, e2, e3)
    u1, t1, t2, t3 = _tc_merge_support(p, b1, W2, weight_b, npad // 16)
    q = _sc_spmm(t1, t2, t3, e1, e2, e3)
    return _tc_final(u1, q, b2, n // 10, n)


# speedup vs baseline: 1.1000x; 1.1000x over previous
import functools

import jax
import jax.numpy as jnp
from jax import lax
from jax.experimental import pallas as pl
from jax.experimental.pallas import tpu as pltpu
from jax.experimental.pallas import tpu_sc as plsc

NC = 2
NS = 16
L = 16
CH = 128
SLAB = 4


def _spmm_body(npad, nrnd, s1, s2, s3, e1, e2, e3, part,
               acc, gi, rows, gsem):
    c = lax.axis_index("c")
    s = lax.axis_index("s")
    w = c * NS + s
    rpt = npad // NS

    @pl.loop(0, CH)
    def _zero_rows(i):
        for j in range(rows.shape[2] // L):
            rows[0, i, j * L:(j + 1) * L] = jnp.zeros((L,), jnp.float32)

    for k in range(rpt // CH):
        pltpu.sync_copy(rows.at[0], acc.at[pl.ds(s * rpt + k * CH, CH)])
    plsc.subcore_barrier()

    streams = ((0, e1, s1), (1, e1, s1), (0, e2, s2), (1, e2, s2),
               (0, e3, s3), (1, e3, s3))

    assert nrnd % 2 == 0

    for d, e_ref, s_ref in streams:
        base = w * nrnd

        def _idx_load(i, p, _e=e_ref):
            pltpu.sync_copy(_e.at[:, pl.ds((base + i) * CH, CH)], gi.at[p])

        def _gather(i, p, rb, _s=s_ref, _d=d):
            del i
            pltpu.async_copy(_s.at[gi.at[p, _d]], rows.at[rb], gsem.at[rb])

        def _gwait(rb, _s=s_ref, _d=d):
            pltpu.make_async_copy(_s.at[gi.at[0, _d]], rows.at[rb],
                                  gsem.at[rb]).wait()

        def _scatter(i, p, rb, _d=d):
            pltpu.sync_copy(rows.at[rb], acc.at[gi.at[p, 1 - _d]], add=True)

        _idx_load(0, 0)
        _gather(0, 0, 0)

        @pl.loop(0, nrnd // 2)
        def _two_chunks(t, _idx_load=_idx_load, _gather=_gather,
                        _gwait=_gwait, _scatter=_scatter):
            i0 = 2 * t
            _idx_load(i0 + 1, 1)
            _gather(i0 + 1, 1, 1)
            _gwait(0)
            _scatter(i0, 0, 0)

            @pl.when(i0 + 2 < nrnd)
            def _():
                _idx_load(i0 + 2, 0)
                _gather(i0 + 2, 0, 0)

            _gwait(1)
            _scatter(i0 + 1, 1, 1)

    plsc.subcore_barrier()
    pltpu.sync_copy(acc.at[pl.ds(s * rpt, rpt)],
                    part.at[c, pl.ds(s * rpt, rpt)])


def _sc_spmm(s1, s2, s3, e1, e2, e3):
    npad, d_model = s1.shape
    nchunk = e1.shape[1] // CH
    nrnd = nchunk // (NC * NS)
    mesh = plsc.VectorSubcoreMesh(core_axis_name="c", subcore_axis_name="s")
    body = functools.partial(_spmm_body, npad, nrnd)
    return pl.kernel(
        body,
        out_type=jax.ShapeDtypeStruct((NC, npad, d_model), jnp.float32),
        mesh=mesh,
        scratch_types=[
            pltpu.VMEM_SHARED((npad, d_model), jnp.float32),
            pltpu.VMEM((2, 2, CH), jnp.int32),
            pltpu.VMEM((2, CH, d_model), jnp.float32),
            pltpu.SemaphoreType.DMA((2,)),
        ],
    )(s1, s2, s3, e1, e2, e3)


def _tc_scaled_support_body(x_ref, w_ref, wb_ref, s1_ref, s2_ref, s3_ref):
    sup = jnp.dot(x_ref[...], w_ref[...], preferred_element_type=jnp.float32)
    s1_ref[...] = wb_ref[0, 0] * sup
    s2_ref[...] = wb_ref[1, 0] * sup
    s3_ref[...] = wb_ref[2, 0] * sup


def _tc_scaled_support(x, w, wb, bm):
    n, d_model = x.shape
    grid = (n // bm,)
    blk = pl.BlockSpec((bm, d_model), lambda i: (i, 0))
    return pl.pallas_call(
        _tc_scaled_support_body,
        grid=grid,
        in_specs=[blk,
                  pl.BlockSpec((d_model, d_model), lambda i: (0, 0)),
                  pl.BlockSpec(memory_space=pltpu.SMEM)],
        out_specs=[blk, blk, blk],
        out_shape=[jax.ShapeDtypeStruct((n, d_model), jnp.float32)] * 3,
    )(x, w, wb)


def _tc_merge_support_body(p_ref, b_ref, w_ref, wb_ref,
                           u_ref, s1_ref, s2_ref, s3_ref):
    u = p_ref[0] + p_ref[1] + b_ref[...]
    u_ref[...] = u
    sup = jnp.dot(u, w_ref[...], preferred_element_type=jnp.float32)
    s1_ref[...] = wb_ref[0, 0] * sup
    s2_ref[...] = wb_ref[1, 0] * sup
    s3_ref[...] = wb_ref[2, 0] * sup


def _tc_merge_support(p, b, w, wb, bm):
    _, n, d_model = p.shape
    grid = (n // bm,)
    blk = pl.BlockSpec((bm, d_model), lambda i: (i, 0))
    return pl.pallas_call(
        _tc_merge_support_body,
        grid=grid,
        in_specs=[pl.BlockSpec((2, bm, d_model), lambda i: (0, i, 0)),
                  pl.BlockSpec((1, d_model), lambda i: (0, 0)),
                  pl.BlockSpec((d_model, d_model), lambda i: (0, 0)),
                  pl.BlockSpec(memory_space=pltpu.SMEM)],
        out_specs=[blk, blk, blk, blk],
        out_shape=[jax.ShapeDtypeStruct((n, d_model), jnp.float32)] * 4,
    )(p, b.reshape(1, d_model), w, wb)


def _tc_final_body(u1_ref, q_ref, b_ref, out_ref):
    out_ref[...] = 0.5 * (u1_ref[...] + q_ref[0] + q_ref[1] + b_ref[...])


def _tc_final(u1, q, b, bm, n):
    _, d_model = u1.shape
    grid = (n // bm,)
    blk = pl.BlockSpec((bm, d_model), lambda i: (i, 0))
    return pl.pallas_call(
        _tc_final_body,
        grid=grid,
        in_specs=[blk,
                  pl.BlockSpec((2, bm, d_model), lambda i: (0, i, 0)),
                  pl.BlockSpec((1, d_model), lambda i: (0, 0))],
        out_specs=blk,
        out_shape=jax.ShapeDtypeStruct((n, d_model), jnp.float32),
    )(u1, q, b.reshape(1, d_model))


def _pad_edges(e, n, nchunk_pad):
    e = e.astype(jnp.int32)
    pad = nchunk_pad * CH - e.shape[1]
    return jnp.pad(e, ((0, 0), (0, pad)), constant_values=n)


def kernel(x, edge_index1, edge_index2, edge_index3, weight_b, W1, b1, W2, b2):
    n, d_model = x.shape
    npad = ((n + NS * CH - 1) // (NS * CH)) * (NS * CH)
    e = edge_index1.shape[1]
    echunk = NC * NS * 2 * CH
    nchunk_pad = ((e + echunk - 1) // echunk) * echunk // CH
    e1 = _pad_edges(edge_index1, n, nchunk_pad)
    e2 = _pad_edges(edge_index2, n, nchunk_pad)
    e3 = _pad_edges(edge_index3, n, nchunk_pad)
    x_pad = jnp.pad(x, ((0, npad - n), (0, 0)))

    s1, s2, s3 = _tc_scaled_support(x_pad, W1, weight_b, npad // 16)
    p = _sc_spmm(s1, s2, s3, e1, e2, e3)
    u1, t1, t2, t3 = _tc_merge_support(p, b1, W2, weight_b, npad // 16)
    q = _sc_spmm(t1, t2, t3, e1, e2, e3)
    return _tc_final(u1, q, b2, n // 10, n)
